# Initial kernel scaffold; baseline (speedup 1.0000x reference)
import functools
import jax
import jax.numpy as jnp
from jax import lax
from jax.experimental import pallas as pl
from jax.experimental.pallas import tpu as pltpu
from jax.experimental.pallas import tpu_sc as plsc

N = 10000
G = 16
IN = 128
HID = 64
OUT = 128
NEG_A = 0.01
NEG_G = 0.2

NP = 10240
E = 320000
ET = E + N
PAD_E = 335872
EROWS = PAD_E // 128
NSUB = 16
PT_ROWS = EROWS // NSUB
P1_CH = PT_ROWS // 4
NPT = NP // NSUB

_f32 = jnp.float32
_i32 = jnp.int32


def _lrelu(x, s):
    return jnp.where(x > 0, x, x * s)


def _splat(v, lane):
    idx = jnp.full((16, 1), lane, dtype=_i32)
    dn = lax.GatherDimensionNumbers(
        offset_dims=(), collapsed_slice_dims=(0,), start_index_map=(0,))
    return lax.gather(v, idx, dn, slice_sizes=(1,),
                      mode=lax.GatherScatterMode.PROMISE_IN_BOUNDS)


def _counts_body(b_ref, cnt_ref, fi_ref):
    b = b_ref[...]
    for g in range(G):
        cnt = jnp.sum((b == g).astype(_f32))
        fi = jnp.sum((b < g).astype(_i32))
        fi = jnp.minimum(fi, N - 1)
        cnt_ref[pl.ds(g, 1), :] = jnp.full((1, 128), cnt, _f32)
        fi_ref[pl.ds(g, 1), :] = jnp.full((1, 128), fi, _i32)


def _counts(batch2d):
    return pl.pallas_call(
        _counts_body,
        out_shape=(jax.ShapeDtypeStruct((G, 128), _f32),
                   jax.ShapeDtypeStruct((G, 128), _i32)),
    )(batch2d)


_R1 = 512


def _dense1_body(x_ref, we_ref, be_ref, l1_ref, aa_ref, a_ref, asad_ref):
    h0 = jnp.dot(x_ref[...], we_ref[...], preferred_element_type=_f32)
    h0 = _lrelu(h0 + be_ref[...], NEG_A)
    a = jnp.dot(h0, l1_ref[...], preferred_element_type=_f32)
    a_ref[...] = a
    h1 = a[:, :HID]
    asad_ref[...] = lax.dot_general(
        aa_ref[...], h1, (((1,), (1,)), ((), ())),
        preferred_element_type=_f32)


def _dense1(xp, W_emb, b_emb, l1cat, aa1):
    grid = NP // _R1
    return pl.pallas_call(
        _dense1_body,
        grid=(grid,),
        in_specs=[
            pl.BlockSpec((_R1, IN), lambda i: (i, 0)),
            pl.BlockSpec((IN, HID), lambda i: (0, 0)),
            pl.BlockSpec((1, HID), lambda i: (0, 0)),
            pl.BlockSpec((HID, 128), lambda i: (0, 0)),
            pl.BlockSpec((8, HID), lambda i: (0, 0)),
        ],
        out_specs=(
            pl.BlockSpec((_R1, 128), lambda i: (i, 0)),
            pl.BlockSpec((8, _R1), lambda i: (0, i)),
        ),
        out_shape=(jax.ShapeDtypeStruct((NP, 128), _f32),
                   jax.ShapeDtypeStruct((8, NP), _f32)),
    )(xp, W_emb, b_emb, l1cat, aa1)


def _dense2_body(pl_ref, ph_ref, b_ref, l2_ref, aa_ref, a_ref, asad_ref):
    s = jnp.concatenate([pl_ref[...], ph_ref[...]], axis=1)
    hin = _lrelu(s + b_ref[...], NEG_A)
    a = jnp.dot(hin, l2_ref[...], preferred_element_type=_f32)
    a_ref[...] = a
    h2 = a[:, :HID]
    asad_ref[...] = lax.dot_general(
        aa_ref[...], h2, (((1,), (1,)), ((), ())),
        preferred_element_type=_f32)


def _dense2(p_lo, p_hi, b1, l2cat, aa2):
    grid = NP // _R1
    return pl.pallas_call(
        _dense2_body,
        grid=(grid,),
        in_specs=[
            pl.BlockSpec((_R1, 32), lambda i: (i, 0)),
            pl.BlockSpec((_R1, 32), lambda i: (i, 0)),
            pl.BlockSpec((1, HID), lambda i: (0, 0)),
            pl.BlockSpec((HID, 128), lambda i: (0, 0)),
            pl.BlockSpec((8, HID), lambda i: (0, 0)),
        ],
        out_specs=(
            pl.BlockSpec((_R1, 128), lambda i: (i, 0)),
            pl.BlockSpec((8, _R1), lambda i: (0, i)),
        ),
        out_shape=(jax.ShapeDtypeStruct((NP, 128), _f32),
                   jax.ShapeDtypeStruct((8, NP), _f32)),
    )(p_lo, p_hi, b1, l2cat, aa2)


def _readout_body(fi_ref, pl_ref, ph_ref, b2_ref, wh_ref, bh_ref, o_ref):
    rows = []
    for g in range(G):
        i = fi_ref[g]
        rows.append(jnp.concatenate(
            [pl_ref[pl.ds(i, 1), :], ph_ref[pl.ds(i, 1), :]], axis=1))
    r = jnp.concatenate(rows, axis=0)
    r = _lrelu(r + b2_ref[...], NEG_A)
    o_ref[...] = jnp.dot(r, wh_ref[...], preferred_element_type=_f32) \
        + bh_ref[...]


def _readout(fi, p_lo, p_hi, b2, W_head, b_head):
    return pl.pallas_call(
        _readout_body,
        grid_spec=pltpu.PrefetchScalarGridSpec(
            num_scalar_prefetch=1,
            grid=(1,),
            in_specs=[
                pl.BlockSpec((NP, 32), lambda i, fi: (0, 0)),
                pl.BlockSpec((NP, 32), lambda i, fi: (0, 0)),
                pl.BlockSpec((1, HID), lambda i, fi: (0, 0)),
                pl.BlockSpec((HID, OUT), lambda i, fi: (0, 0)),
                pl.BlockSpec((1, OUT), lambda i, fi: (0, 0)),
            ],
            out_specs=pl.BlockSpec((G, OUT), lambda i, fi: (0, 0)),
        ),
        out_shape=jax.ShapeDtypeStruct((G, OUT), _f32),
    )(fi, p_lo, p_hi, b2, W_head, b_head)


def _edge_body(a4, asad, src2, dst2, zrows, zvec, out_hbm,
               as_l, ad_l, den_l, e_buf, src_c, dst_c, idx_c, rows_c,
               den_sp, out_sp, gsem, ssem):
    cid = lax.axis_index("c")
    sid = lax.axis_index("s")

    pltpu.sync_copy(zrows.at[pl.ds(sid * NPT, NPT), :],
                    out_sp.at[pl.ds(sid * NPT, NPT), :])
    pltpu.sync_copy(zvec.at[pl.ds(sid * NPT, NPT)],
                    den_sp.at[pl.ds(sid * NPT, NPT)])
    pltpu.sync_copy(asad.at[0], as_l)
    pltpu.sync_copy(asad.at[1], ad_l)
    plsc.subcore_barrier()

    row_base = sid * PT_ROWS

    def p1_chunk(i, carry):
        r0 = row_base + i * 4
        pltpu.sync_copy(src2.at[pl.ds(r0, 4), :], src_c)
        pltpu.sync_copy(dst2.at[pl.ds(r0, 4), :], dst_c)
        for q in range(4):
            for l in range(8):
                sv = src_c[q, pl.ds(l * 16, 16)]
                dv = dst_c[q, pl.ds(l * 16, 16)]
                al = plsc.load_gather(as_l, [sv]) \
                    + plsc.load_gather(ad_l, [dv])
                ev = jnp.exp(_lrelu(al, NEG_G))
                e_buf[pl.ds(i * 512 + q * 128 + l * 16, 16)] = ev
        descs = []
        for q in range(4):
            descs.append(pltpu.async_copy(
                e_buf.at[pl.ds(i * 512 + q * 128, 128)],
                den_sp.at[dst_c.at[q]], ssem, add=True))
        for d in descs:
            d.wait()
        return carry

    lax.fori_loop(0, P1_CH, p1_chunk, 0)
    plsc.subcore_barrier()

    pltpu.sync_copy(den_sp, den_l)

    def p2_chunk(i, carry):
        r0 = row_base + i * 4
        pltpu.sync_copy(src2.at[pl.ds(r0, 4), :], src_c)
        pltpu.sync_copy(dst2.at[pl.ds(r0, 4), :], dst_c)
        for q in range(4):
            for l in range(8):
                sv = src_c[q, pl.ds(l * 16, 16)]
                idx_c[q, pl.ds(l * 16, 16)] = sv * 4 + cid
        gd = []
        for q in range(4):
            gd.append(pltpu.async_copy(
                a4.at[idx_c.at[q]],
                rows_c.at[pl.ds(q * 128, 128), :], gsem))
        for d in gd:
            d.wait()
        for q in range(4):
            for l in range(8):
                ev = e_buf[pl.ds(i * 512 + q * 128 + l * 16, 16)]
                dv = dst_c[q, pl.ds(l * 16, 16)]
                den = plsc.load_gather(den_l, [dv]) + 1e-16
                wv = ev / den
                for t in range(16):
                    ei = q * 128 + l * 16 + t
                    ws = _splat(wv, t)
                    rows_c[ei, pl.ds(0, 16)] = rows_c[ei, pl.ds(0, 16)] * ws
                    rows_c[ei, pl.ds(16, 16)] = rows_c[ei, pl.ds(16, 16)] * ws
        sd = []
        for q in range(4):
            sd.append(pltpu.async_copy(
                rows_c.at[pl.ds(q * 128, 128), :],
                out_sp.at[dst_c.at[q]], ssem, add=True))
        for d in sd:
            d.wait()
        return carry

    lax.fori_loop(0, P1_CH, p2_chunk, 0)
    plsc.subcore_barrier()

    pltpu.sync_copy(out_sp.at[pl.ds(sid * NPT, NPT), :],
                    out_hbm.at[cid, pl.ds(sid * NPT, NPT), :])


@functools.partial(
    pl.kernel,
    out_type=jax.ShapeDtypeStruct((2, NP, 32), _f32),
    mesh=plsc.VectorSubcoreMesh(core_axis_name="c", subcore_axis_name="s"),
    scratch_types=[
        pltpu.VMEM((NP,), _f32),
        pltpu.VMEM((NP,), _f32),
        pltpu.VMEM((NP,), _f32),
        pltpu.VMEM((PAD_E // NSUB,), _f32),
        pltpu.VMEM((4, 128), _i32),
        pltpu.VMEM((4, 128), _i32),
        pltpu.VMEM((4, 128), _i32),
        pltpu.VMEM((512, 32), _f32),
        pltpu.VMEM_SHARED((NP,), _f32),
        pltpu.VMEM_SHARED((NP, 32), _f32),
        pltpu.SemaphoreType.DMA,
        pltpu.SemaphoreType.DMA,
    ],
)
def _edge_layer(a4, asad, src2, dst2, zrows, zvec, out_hbm,
                as_l, ad_l, den_l, e_buf, src_c, dst_c, idx_c, rows_c,
                den_sp, out_sp, gsem, ssem):
    _edge_body(a4, asad, src2, dst2, zrows, zvec, out_hbm,
               as_l, ad_l, den_l, e_buf, src_c, dst_c, idx_c, rows_c,
               den_sp, out_sp, gsem, ssem)


def kernel(x, edge_index, batch, W_emb, b_emb, lin1, a1s, a1d, b1,
           lin2, a2s, a2d, b2, W_head, b_head):
    xp = jnp.pad(x, ((0, NP - N), (0, 0)))
    batch_p = jnp.pad(batch, (0, NP - N), constant_values=G)

    loop = jnp.arange(N, dtype=_i32)
    npad = PAD_E - ET
    src = jnp.concatenate(
        [edge_index[0], loop, jnp.zeros((npad,), _i32)])
    dst = jnp.concatenate(
        [edge_index[1], loop,
         N + (jnp.arange(npad, dtype=_i32) % (NP - N))])
    src2 = src.reshape(EROWS, 128)
    dst2 = dst.reshape(EROWS, 128)

    z6 = jnp.zeros((6, HID), _f32)
    l1cat = jnp.concatenate(
        [lin1, a1s[:, None], a1d[:, None], jnp.zeros((HID, 62), _f32)], 1)
    l2cat = jnp.concatenate(
        [lin2, a2s[:, None], a2d[:, None], jnp.zeros((HID, 62), _f32)], 1)
    aa1 = jnp.concatenate([a1s[None], a1d[None], z6], 0)
    aa2 = jnp.concatenate([a2s[None], a2d[None], z6], 0)

    zrows = jnp.zeros((NP, 32), _f32)
    zvec = jnp.zeros((NP,), _f32)

    cnt, fi = _counts(batch_p.reshape(80, 128))

    a1_, asad1 = _dense1(xp, W_emb, b_emb.reshape(1, HID), l1cat, aa1)
    p1 = _edge_layer(a1_.reshape(NP * 4, 32), asad1, src2, dst2, zrows, zvec)

    a2_, asad2 = _dense2(p1[0], p1[1], b1.reshape(1, HID), l2cat, aa2)
    p2 = _edge_layer(a2_.reshape(NP * 4, 32), asad2, src2, dst2, zrows, zvec)

    out = _readout(fi[:, 0], p2[0], p2[1], b2.reshape(1, HID),
                   W_head, b_head.reshape(1, OUT))
    num_nodes = cnt[:, :1]
    return (out, num_nodes)


# speedup vs baseline: 27.3068x; 27.3068x over previous
import functools
import jax
import jax.numpy as jnp
from jax import lax
from jax.experimental import pallas as pl
from jax.experimental.pallas import tpu as pltpu
from jax.experimental.pallas import tpu_sc as plsc

N = 10000
G = 16
IN = 128
HID = 64
OUT = 128
NEG_A = 0.01
NEG_G = 0.2

NP = 10240
E = 320000
ET = E + N
PAD_E = 335872
EROWS = PAD_E // 128
NSUB = 16
PT_ROWS = EROWS // NSUB
P1_CH = PT_ROWS // 4
NPT = NP // NSUB

_f32 = jnp.float32
_i32 = jnp.int32


def _lrelu(x, s):
    return jnp.where(x > 0, x, x * s)


def _splat(v, lane):
    idx = jnp.full((16, 1), lane, dtype=_i32)
    dn = lax.GatherDimensionNumbers(
        offset_dims=(), collapsed_slice_dims=(0,), start_index_map=(0,))
    return lax.gather(v, idx, dn, slice_sizes=(1,),
                      mode=lax.GatherScatterMode.PROMISE_IN_BOUNDS)


def _counts_body(b_ref, cnt_ref, fi_ref):
    b = b_ref[...]
    for g in range(G):
        cnt = jnp.sum((b == g).astype(_f32))
        fi = jnp.sum((b < g).astype(_i32))
        fi = jnp.minimum(fi, N - 1)
        cnt_ref[pl.ds(g, 1), :] = jnp.full((1, 128), cnt, _f32)
        fi_ref[pl.ds(g, 1), :] = jnp.full((1, 128), fi, _i32)


def _counts(batch2d):
    return pl.pallas_call(
        _counts_body,
        out_shape=(jax.ShapeDtypeStruct((G, 128), _f32),
                   jax.ShapeDtypeStruct((G, 128), _i32)),
    )(batch2d)


_R1 = 512


def _dense1_body(x_ref, we_ref, be_ref, l1_ref, aa_ref, a_ref, asad_ref):
    h0 = jnp.dot(x_ref[...], we_ref[...], preferred_element_type=_f32)
    h0 = _lrelu(h0 + be_ref[...], NEG_A)
    a = jnp.dot(h0, l1_ref[...], preferred_element_type=_f32)
    a_ref[...] = a
    h1 = a[:, :HID]
    asad_ref[...] = lax.dot_general(
        aa_ref[...], h1, (((1,), (1,)), ((), ())),
        preferred_element_type=_f32)


def _dense1(xp, W_emb, b_emb, l1cat, aa1):
    grid = NP // _R1
    return pl.pallas_call(
        _dense1_body,
        grid=(grid,),
        in_specs=[
            pl.BlockSpec((_R1, IN), lambda i: (i, 0)),
            pl.BlockSpec((IN, HID), lambda i: (0, 0)),
            pl.BlockSpec((1, HID), lambda i: (0, 0)),
            pl.BlockSpec((HID, 128), lambda i: (0, 0)),
            pl.BlockSpec((8, HID), lambda i: (0, 0)),
        ],
        out_specs=(
            pl.BlockSpec((_R1, 128), lambda i: (i, 0)),
            pl.BlockSpec((8, _R1), lambda i: (0, i)),
        ),
        out_shape=(jax.ShapeDtypeStruct((NP, 128), _f32),
                   jax.ShapeDtypeStruct((8, NP), _f32)),
    )(xp, W_emb, b_emb, l1cat, aa1)


def _dense2_body(pl_ref, ph_ref, b_ref, l2_ref, aa_ref, a_ref, asad_ref):
    s = jnp.concatenate([pl_ref[...], ph_ref[...]], axis=1)
    hin = _lrelu(s + b_ref[...], NEG_A)
    a = jnp.dot(hin, l2_ref[...], preferred_element_type=_f32)
    a_ref[...] = a
    h2 = a[:, :HID]
    asad_ref[...] = lax.dot_general(
        aa_ref[...], h2, (((1,), (1,)), ((), ())),
        preferred_element_type=_f32)


def _dense2(p_lo, p_hi, b1, l2cat, aa2):
    grid = NP // _R1
    return pl.pallas_call(
        _dense2_body,
        grid=(grid,),
        in_specs=[
            pl.BlockSpec((_R1, 32), lambda i: (i, 0)),
            pl.BlockSpec((_R1, 32), lambda i: (i, 0)),
            pl.BlockSpec((1, HID), lambda i: (0, 0)),
            pl.BlockSpec((HID, 128), lambda i: (0, 0)),
            pl.BlockSpec((8, HID), lambda i: (0, 0)),
        ],
        out_specs=(
            pl.BlockSpec((_R1, 128), lambda i: (i, 0)),
            pl.BlockSpec((8, _R1), lambda i: (0, i)),
        ),
        out_shape=(jax.ShapeDtypeStruct((NP, 128), _f32),
                   jax.ShapeDtypeStruct((8, NP), _f32)),
    )(p_lo, p_hi, b1, l2cat, aa2)


def _readout_body(fi_ref, pl_ref, ph_ref, b2_ref, wh_ref, bh_ref, o_ref):
    rows = []
    for g in range(G):
        i = fi_ref[g]
        rows.append(jnp.concatenate(
            [pl_ref[pl.ds(i, 1), :], ph_ref[pl.ds(i, 1), :]], axis=1))
    r = jnp.concatenate(rows, axis=0)
    r = _lrelu(r + b2_ref[...], NEG_A)
    o_ref[...] = jnp.dot(r, wh_ref[...], preferred_element_type=_f32) \
        + bh_ref[...]


def _readout(fi, p_lo, p_hi, b2, W_head, b_head):
    return pl.pallas_call(
        _readout_body,
        grid_spec=pltpu.PrefetchScalarGridSpec(
            num_scalar_prefetch=1,
            grid=(1,),
            in_specs=[
                pl.BlockSpec((NP, 32), lambda i, fi: (0, 0)),
                pl.BlockSpec((NP, 32), lambda i, fi: (0, 0)),
                pl.BlockSpec((1, HID), lambda i, fi: (0, 0)),
                pl.BlockSpec((HID, OUT), lambda i, fi: (0, 0)),
                pl.BlockSpec((1, OUT), lambda i, fi: (0, 0)),
            ],
            out_specs=pl.BlockSpec((G, OUT), lambda i, fi: (0, 0)),
        ),
        out_shape=jax.ShapeDtypeStruct((G, OUT), _f32),
    )(fi, p_lo, p_hi, b2, W_head, b_head)


def _edge_body(a4, asad, src2, dst2, zrows, zvec, out_hbm,
               as_l, ad_l, den_l, e_buf, src_c, dst_c, idx_c, rows_c,
               den_sp, out_sp, gsem, ssem):
    cid = lax.axis_index("c")
    sid = lax.axis_index("s")

    pltpu.sync_copy(zrows.at[pl.ds(sid * NPT, NPT), :],
                    out_sp.at[pl.ds(sid * NPT, NPT), :])
    pltpu.sync_copy(zvec.at[pl.ds(sid * NPT, NPT)],
                    den_sp.at[pl.ds(sid * NPT, NPT)])
    pltpu.sync_copy(asad.at[0], as_l)
    pltpu.sync_copy(asad.at[1], ad_l)
    plsc.subcore_barrier()

    row_base = sid * PT_ROWS

    def p1_chunk(i, carry):
        r0 = row_base + i * 4
        pltpu.sync_copy(src2.at[pl.ds(r0, 4), :], src_c)
        pltpu.sync_copy(dst2.at[pl.ds(r0, 4), :], dst_c)
        for q in range(4):
            for l in range(8):
                sv = src_c[q, pl.ds(l * 16, 16)]
                dv = dst_c[q, pl.ds(l * 16, 16)]
                al = plsc.load_gather(as_l, [sv]) \
                    + plsc.load_gather(ad_l, [dv])
                ev = jnp.exp(_lrelu(al, NEG_G))
                e_buf[pl.ds(i * 512 + q * 128 + l * 16, 16)] = ev
        descs = []
        for q in range(4):
            descs.append(pltpu.async_copy(
                e_buf.at[pl.ds(i * 512 + q * 128, 128)],
                den_sp.at[dst_c.at[q]], ssem, add=True))
        for d in descs:
            d.wait()
        return carry

    lax.fori_loop(0, P1_CH, p1_chunk, 0)
    plsc.subcore_barrier()

    pltpu.sync_copy(den_sp, den_l)

    def p2_chunk(i, carry):
        r0 = row_base + i * 4
        pltpu.sync_copy(src2.at[pl.ds(r0, 4), :], src_c)
        pltpu.sync_copy(dst2.at[pl.ds(r0, 4), :], dst_c)
        for q in range(4):
            for l in range(8):
                sv = src_c[q, pl.ds(l * 16, 16)]
                idx_c[q, pl.ds(l * 16, 16)] = sv * 4 + cid
        gd = []
        for q in range(4):
            gd.append(pltpu.async_copy(
                a4.at[idx_c.at[q]],
                rows_c.at[pl.ds(q * 128, 128), :], gsem))
        for d in gd:
            d.wait()
        for q in range(4):
            for l in range(8):
                ev = e_buf[pl.ds(i * 512 + q * 128 + l * 16, 16)]
                dv = dst_c[q, pl.ds(l * 16, 16)]
                den = plsc.load_gather(den_l, [dv]) + 1e-16
                wv = ev / den
                for t in range(16):
                    ei = q * 128 + l * 16 + t
                    ws = _splat(wv, t)
                    rows_c[ei, pl.ds(0, 16)] = rows_c[ei, pl.ds(0, 16)] * ws
                    rows_c[ei, pl.ds(16, 16)] = rows_c[ei, pl.ds(16, 16)] * ws
        sd = []
        for q in range(4):
            sd.append(pltpu.async_copy(
                rows_c.at[pl.ds(q * 128, 128), :],
                out_sp.at[dst_c.at[q]], ssem, add=True))
        for d in sd:
            d.wait()
        return carry

    lax.fori_loop(0, P1_CH, p2_chunk, 0)
    plsc.subcore_barrier()

    pltpu.sync_copy(out_sp.at[pl.ds(sid * NPT, NPT), :],
                    out_hbm.at[cid, pl.ds(sid * NPT, NPT), :])


@functools.cache
def _edge_layer_fn():
  return functools.partial(
    pl.kernel,
    out_type=jax.ShapeDtypeStruct((2, NP, 32), _f32),
    mesh=plsc.VectorSubcoreMesh(core_axis_name="c", subcore_axis_name="s",
                                num_cores=2, num_subcores=NSUB),
    scratch_types=[
        pltpu.VMEM((NP,), _f32),
        pltpu.VMEM((NP,), _f32),
        pltpu.VMEM((NP,), _f32),
        pltpu.VMEM((PAD_E // NSUB,), _f32),
        pltpu.VMEM((4, 128), _i32),
        pltpu.VMEM((4, 128), _i32),
        pltpu.VMEM((4, 128), _i32),
        pltpu.VMEM((512, 32), _f32),
        pltpu.VMEM_SHARED((NP,), _f32),
        pltpu.VMEM_SHARED((NP, 32), _f32),
        pltpu.SemaphoreType.DMA,
        pltpu.SemaphoreType.DMA,
    ],
    compiler_params=pltpu.CompilerParams(needs_layout_passes=False,
                                         use_tc_tiling_on_sc=False),
  )(_edge_body)


def _edge_layer(a4, asad, src2, dst2, zrows, zvec):
    return _edge_layer_fn()(a4, asad, src2, dst2, zrows, zvec)


def kernel(x, edge_index, batch, W_emb, b_emb, lin1, a1s, a1d, b1,
           lin2, a2s, a2d, b2, W_head, b_head):
    xp = jnp.pad(x, ((0, NP - N), (0, 0)))
    batch_p = jnp.pad(batch, (0, NP - N), constant_values=G)

    loop = jnp.arange(N, dtype=_i32)
    npad = PAD_E - ET
    src = jnp.concatenate(
        [edge_index[0], loop, jnp.zeros((npad,), _i32)])
    dst = jnp.concatenate(
        [edge_index[1], loop,
         N + (jnp.arange(npad, dtype=_i32) % (NP - N))])
    src2 = src.reshape(EROWS, 128)
    dst2 = dst.reshape(EROWS, 128)

    z6 = jnp.zeros((6, HID), _f32)
    l1cat = jnp.concatenate(
        [lin1, a1s[:, None], a1d[:, None], jnp.zeros((HID, 62), _f32)], 1)
    l2cat = jnp.concatenate(
        [lin2, a2s[:, None], a2d[:, None], jnp.zeros((HID, 62), _f32)], 1)
    aa1 = jnp.concatenate([a1s[None], a1d[None], z6], 0)
    aa2 = jnp.concatenate([a2s[None], a2d[None], z6], 0)

    zrows = jnp.zeros((NP, 32), _f32)
    zvec = jnp.zeros((NP,), _f32)

    cnt, fi = _counts(batch_p.reshape(80, 128))

    a1_, asad1 = _dense1(xp, W_emb, b_emb.reshape(1, HID), l1cat, aa1)
    p1 = _edge_layer(a1_.reshape(NP * 4, 32), asad1, src2, dst2, zrows, zvec)

    a2_, asad2 = _dense2(p1[0], p1[1], b1.reshape(1, HID), l2cat, aa2)
    p2 = _edge_layer(a2_.reshape(NP * 4, 32), asad2, src2, dst2, zrows, zvec)

    out = _readout(fi[:, 0], p2[0], p2[1], b2.reshape(1, HID),
                   W_head, b_head.reshape(1, OUT))
    num_nodes = cnt[:, :1]
    return (out, num_nodes)
